# baseline (speedup 1.0000x reference)
import functools
import math

import jax
import jax.numpy as jnp
from jax import lax
from jax.experimental import pallas as pl
from jax.experimental.pallas import tpu as pltpu
from jax.experimental.pallas import tpu_sc as plsc

_D = 64
_SCALE = float(math.sqrt(_D))
_NC, _NS = 2, 16
_NW = _NC * _NS
_CHUNK = 128
_K = 2
_GROUP = _K * _CHUNK
_NBUF = 4


@functools.lru_cache(maxsize=None)
def _make_kernel(B):
    assert B % (_NW * _GROUP) == 0
    b_per_w = B // _NW
    n_chunks = b_per_w // _CHUNK
    n_groups = n_chunks // _K
    assert n_groups % _NBUF == 0

    mesh = plsc.VectorSubcoreMesh(core_axis_name="c", subcore_axis_name="s")

    @functools.partial(
        pl.kernel,
        out_type=jax.ShapeDtypeStruct((B, _D), jnp.float32),
        mesh=mesh,
        compiler_params=pltpu.CompilerParams(use_tc_tiling_on_sc=False),
        scratch_types=[
            pltpu.VMEM((n_chunks, _CHUNK), jnp.int32),
            [pltpu.VMEM((_GROUP, _D), jnp.float32) for _ in range(_NBUF)],
            [pltpu.SemaphoreType.DMA for _ in range(_NBUF)],
            [pltpu.SemaphoreType.DMA for _ in range(_NBUF)],
        ],
    )
    def emb_kernel(x_hbm, lut_hbm, out_hbm, idx_v, bufs, gsems, osems):
        wid = lax.axis_index("s") * _NC + lax.axis_index("c")
        base = wid * b_per_w

        pltpu.sync_copy(x_hbm.at[wid], idx_v)

        def fire_gathers(g, b):
            for kk in range(_K):
                pltpu.async_copy(
                    lut_hbm.at[idx_v.at[g * _K + kk]],
                    bufs[b].at[pl.ds(kk * _CHUNK, _CHUNK)],
                    gsems[b],
                )

        def drain_gathers(b):
            pltpu.make_async_copy(
                out_hbm.at[pl.ds(base, _GROUP)], bufs[b], gsems[b]
            ).wait()

        def wait_out(b):
            pltpu.make_async_copy(
                bufs[b], out_hbm.at[pl.ds(base, _GROUP)], osems[b]
            ).wait()

        fire_gathers(0, 0)
        fire_gathers(1, 1)

        @pl.loop(0, n_groups // _NBUF)
        def _(i):
            for b in range(_NBUF):
                g = i * _NBUF + b
                drain_gathers(b)

                @pl.loop(0, _GROUP)
                def _(r):
                    for j in range(_D // 16):
                        sl = pl.ds(j * 16, 16)
                        bufs[b][r, sl] = bufs[b][r, sl] * _SCALE

                pltpu.async_copy(
                    bufs[b],
                    out_hbm.at[pl.ds(base + g * _GROUP, _GROUP)],
                    osems[b],
                )
                nb = (b + 2) % _NBUF
                ng = g + 2

                @pl.when(ng < n_groups)
                def _():
                    @pl.when(ng >= _NBUF)
                    def _():
                        wait_out(nb)

                    fire_gathers(ng, nb)

        for b in range(_NBUF):
            wait_out(b)

    return emb_kernel


def kernel(x, lut):
    rows, cols = x.shape
    B = rows * cols
    xr = x.reshape(_NW, B // (_NW * _CHUNK), _CHUNK)
    out = _make_kernel(B)(xr, lut)
    return out.reshape(rows, cols, _D)
